# baseline (speedup 1.0000x reference)
import jax
import jax.numpy as jnp
from jax.experimental import pallas as pl
from jax.experimental.pallas import tpu as pltpu

B, D, H, V = 32, 128, 256, 100000
VBLK = 4096
NBLK = (V + VBLK - 1) // VBLK


def _body(state_ref, ts_ref, w1_ref, b1_ref, w2a_ref, w2b_ref, w2c_ref,
          w2d_ref, b2_ref, samp_out, gath_out, h_ref, m_ref, s_ref, g_ref):
    v = pl.program_id(0)
    nv = pl.num_programs(0)

    @pl.when(v == 0)
    def _init():
        h = jnp.dot(state_ref[...], w1_ref[...],
                    preferred_element_type=jnp.float32)
        h_ref[...] = jnp.maximum(h + b1_ref[...], 0.0)
        m_ref[...] = jnp.full((B, 1), -jnp.inf, jnp.float32)
        s_ref[...] = jnp.zeros((B, 1), jnp.float32)
        g_ref[...] = jnp.zeros((B, 1), jnp.float32)
        samp_out[...] = ts_ref[...]

    h = h_ref[...]
    logits = (jnp.dot(h[:, 0:64], w2a_ref[...],
                      preferred_element_type=jnp.float32)
              + jnp.dot(h[:, 64:128], w2b_ref[...],
                        preferred_element_type=jnp.float32)
              + jnp.dot(h[:, 128:192], w2c_ref[...],
                        preferred_element_type=jnp.float32)
              + jnp.dot(h[:, 192:256], w2d_ref[...],
                        preferred_element_type=jnp.float32)
              + b2_ref[...])
    col = v * VBLK + jax.lax.broadcasted_iota(jnp.int32, (B, VBLK), 1)
    logits = jnp.where(col < V, logits, -jnp.inf)

    bm = jnp.max(logits, axis=1, keepdims=True)
    new_m = jnp.maximum(m_ref[...], bm)
    s_ref[...] = (s_ref[...] * jnp.exp(m_ref[...] - new_m)
                  + jnp.sum(jnp.exp(logits - new_m), axis=1, keepdims=True))
    m_ref[...] = new_m

    hit = col == ts_ref[...]
    g_ref[...] += jnp.sum(jnp.where(hit, logits, 0.0), axis=1, keepdims=True)

    @pl.when(v == nv - 1)
    def _fin():
        gath_out[...] = g_ref[...] - m_ref[...] - jnp.log(s_ref[...])


def kernel(state, true_samples, W1, b1, W2, b2):
    ts = true_samples.astype(jnp.int32)
    b1r = b1.reshape(1, H)
    b2r = b2.reshape(1, V)

    sampled, gathered = pl.pallas_call(
        _body,
        grid=(NBLK,),
        in_specs=[
            pl.BlockSpec((B, D), lambda v: (0, 0)),
            pl.BlockSpec((B, 1), lambda v: (0, 0)),
            pl.BlockSpec((D, H), lambda v: (0, 0)),
            pl.BlockSpec((1, H), lambda v: (0, 0)),
            pl.BlockSpec((H // 4, VBLK), lambda v: (0, v)),
            pl.BlockSpec((H // 4, VBLK), lambda v: (1, v)),
            pl.BlockSpec((H // 4, VBLK), lambda v: (2, v)),
            pl.BlockSpec((H // 4, VBLK), lambda v: (3, v)),
            pl.BlockSpec((1, VBLK), lambda v: (0, v)),
        ],
        out_specs=[
            pl.BlockSpec((B, 1), lambda v: (0, 0)),
            pl.BlockSpec((B, 1), lambda v: (0, 0)),
        ],
        out_shape=[
            jax.ShapeDtypeStruct((B, 1), true_samples.dtype),
            jax.ShapeDtypeStruct((B, 1), jnp.float32),
        ],
        scratch_shapes=[
            pltpu.VMEM((B, H), jnp.float32),
            pltpu.VMEM((B, 1), jnp.float32),
            pltpu.VMEM((B, 1), jnp.float32),
            pltpu.VMEM((B, 1), jnp.float32),
        ],
        compiler_params=pltpu.CompilerParams(
            dimension_semantics=("arbitrary",),
        ),
    )(state, ts, W1, b1r, W2, W2, W2, W2, b2r)

    return (sampled, gathered)


# speedup vs baseline: 1.0635x; 1.0635x over previous
import jax
import jax.numpy as jnp
from jax.experimental import pallas as pl
from jax.experimental.pallas import tpu as pltpu

B, D, H, V = 32, 128, 256, 100000
VBLK = 8192
NBLK = (V + VBLK - 1) // VBLK


def _body(state_ref, ts_ref, w1_ref, b1_ref, w2a_ref, w2b_ref, w2c_ref,
          w2d_ref, b2_ref, samp_out, gath_out, h_ref, m_ref, s_ref, g_ref):
    v = pl.program_id(0)
    nv = pl.num_programs(0)

    @pl.when(v == 0)
    def _init():
        h = jnp.dot(state_ref[...], w1_ref[...],
                    preferred_element_type=jnp.float32)
        h_ref[...] = jnp.maximum(h + b1_ref[...], 0.0)
        m_ref[...] = jnp.full((B, 1), -jnp.inf, jnp.float32)
        s_ref[...] = jnp.zeros((B, 1), jnp.float32)
        g_ref[...] = jnp.zeros((B, 1), jnp.float32)
        samp_out[...] = ts_ref[...]

    h = h_ref[...]
    logits = (jnp.dot(h[:, 0:64], w2a_ref[...],
                      preferred_element_type=jnp.float32)
              + jnp.dot(h[:, 64:128], w2b_ref[...],
                        preferred_element_type=jnp.float32)
              + jnp.dot(h[:, 128:192], w2c_ref[...],
                        preferred_element_type=jnp.float32)
              + jnp.dot(h[:, 192:256], w2d_ref[...],
                        preferred_element_type=jnp.float32)
              + b2_ref[...])
    col = v * VBLK + jax.lax.broadcasted_iota(jnp.int32, (B, VBLK), 1)
    logits = jnp.where(col < V, logits, -jnp.inf)

    bm = jnp.max(logits, axis=1, keepdims=True)
    new_m = jnp.maximum(m_ref[...], bm)
    s_ref[...] = (s_ref[...] * jnp.exp(m_ref[...] - new_m)
                  + jnp.sum(jnp.exp(logits - new_m), axis=1, keepdims=True))
    m_ref[...] = new_m

    hit = col == ts_ref[...]
    g_ref[...] += jnp.sum(jnp.where(hit, logits, 0.0), axis=1, keepdims=True)

    @pl.when(v == nv - 1)
    def _fin():
        gath_out[...] = g_ref[...] - m_ref[...] - jnp.log(s_ref[...])


def kernel(state, true_samples, W1, b1, W2, b2):
    ts = true_samples.astype(jnp.int32)
    b1r = b1.reshape(1, H)
    b2r = b2.reshape(1, V)

    sampled, gathered = pl.pallas_call(
        _body,
        grid=(NBLK,),
        in_specs=[
            pl.BlockSpec((B, D), lambda v: (0, 0)),
            pl.BlockSpec((B, 1), lambda v: (0, 0)),
            pl.BlockSpec((D, H), lambda v: (0, 0)),
            pl.BlockSpec((1, H), lambda v: (0, 0)),
            pl.BlockSpec((H // 4, VBLK), lambda v: (0, v)),
            pl.BlockSpec((H // 4, VBLK), lambda v: (1, v)),
            pl.BlockSpec((H // 4, VBLK), lambda v: (2, v)),
            pl.BlockSpec((H // 4, VBLK), lambda v: (3, v)),
            pl.BlockSpec((1, VBLK), lambda v: (0, v)),
        ],
        out_specs=[
            pl.BlockSpec((B, 1), lambda v: (0, 0)),
            pl.BlockSpec((B, 1), lambda v: (0, 0)),
        ],
        out_shape=[
            jax.ShapeDtypeStruct((B, 1), true_samples.dtype),
            jax.ShapeDtypeStruct((B, 1), jnp.float32),
        ],
        scratch_shapes=[
            pltpu.VMEM((B, H), jnp.float32),
            pltpu.VMEM((B, 1), jnp.float32),
            pltpu.VMEM((B, 1), jnp.float32),
            pltpu.VMEM((B, 1), jnp.float32),
        ],
        compiler_params=pltpu.CompilerParams(
            dimension_semantics=("arbitrary",),
        ),
    )(state, ts, W1, b1r, W2, W2, W2, W2, b2r)

    return (sampled, gathered)
